# Initial kernel scaffold; baseline (speedup 1.0000x reference)
import functools

import jax
import jax.numpy as jnp
from jax import lax
from jax.experimental import pallas as pl
from jax.experimental.pallas import tpu as pltpu
from jax.experimental.pallas import tpu_sc as plsc

N = 10000
E = 320000
D = 128

NC = 2
NS = 16
NW = NC * NS
EPW = E // NW
G = 125
NG = EPW // G
RPS = N // NS
ZCH = 125
NZ = RPS // ZCH


def _sc_body(x_hbm, src_hbm, dst_hbm, dstf_hbm,
             agg_hbm, deg_hbm,
             src_v, dst_v, dstf_v, rows_v, deg_v, acc_sh, sem):
    c = lax.axis_index("c")
    s = lax.axis_index("s")
    w = c * NS + s

    zeros16 = jnp.zeros((16,), jnp.float32)
    ones16 = jnp.ones((16,), jnp.float32)

    def zero_rows(i, _):
        for k in range(D // 16):
            rows_v[i, pl.ds(k * 16, 16)] = zeros16
        return 0
    lax.fori_loop(0, ZCH, zero_rows, 0)

    def zero_deg(i, _):
        deg_v[pl.ds(i * 16, 16)] = zeros16
        return 0
    lax.fori_loop(0, N // 16, zero_deg, 0)

    for j in range(NZ):
        pltpu.sync_copy(rows_v, acc_sh.at[pl.ds(s * RPS + j * ZCH, ZCH), :])

    pltpu.sync_copy(src_hbm.at[w], src_v)
    pltpu.sync_copy(dst_hbm.at[w], dst_v)
    pltpu.sync_copy(dstf_hbm.at[w], dstf_v)

    def deg_step(i, _):
        idx16 = dstf_v[pl.ds(i * 16, 16)]
        plsc.addupdate_scatter(deg_v, [idx16], ones16)
        return 0
    lax.fori_loop(0, EPW // 16, deg_step, 0)

    plsc.subcore_barrier()

    def edge_step(g, _):
        pltpu.async_copy(x_hbm.at[src_v.at[g]], rows_v, sem).wait()
        pltpu.sync_copy(rows_v, acc_sh.at[dst_v.at[g]], add=True)
        return 0
    lax.fori_loop(0, NG, edge_step, 0)

    plsc.subcore_barrier()

    for j in range(NZ):
        base = s * RPS + j * ZCH
        pltpu.sync_copy(acc_sh.at[pl.ds(base, ZCH), :], rows_v)
        pltpu.sync_copy(rows_v, agg_hbm.at[c, pl.ds(base, ZCH), :])
    pltpu.sync_copy(deg_v, deg_hbm.at[w])


@jax.jit
def _sc_aggregate(x, src_r, dst_r, dst_f):
    mesh = plsc.VectorSubcoreMesh(core_axis_name="c", subcore_axis_name="s")
    f = pl.kernel(
        _sc_body,
        out_type=(
            jax.ShapeDtypeStruct((NC, N, D), jnp.float32),
            jax.ShapeDtypeStruct((NW, N), jnp.float32),
        ),
        mesh=mesh,
        scratch_types=[
            pltpu.VMEM((NG, G), jnp.int32),
            pltpu.VMEM((NG, G), jnp.int32),
            pltpu.VMEM((EPW,), jnp.int32),
            pltpu.VMEM((G, D), jnp.float32),
            pltpu.VMEM((N,), jnp.float32),
            pltpu.VMEM_SHARED((N, D), jnp.float32),
            pltpu.SemaphoreType.DMA,
        ],
    )
    return f(x, src_r, dst_r, dst_f)


def _tc_body(agg_ref, deg_ref, x_ref, w_ref, b_ref, o_ref):
    agg = agg_ref[0] + agg_ref[1] + x_ref[...]
    deg = jnp.sum(deg_ref[...], axis=1) + 1.0
    h = agg / deg[:, None]
    o_ref[...] = (
        jnp.dot(h, w_ref[...], preferred_element_type=jnp.float32) + b_ref[...]
    )


def _tc_finish(agg_p, deg_t, x, W, b2):
    blk = 2000
    grid = N // blk
    return pl.pallas_call(
        _tc_body,
        grid=(grid,),
        in_specs=[
            pl.BlockSpec((NC, blk, D), lambda i: (0, i, 0)),
            pl.BlockSpec((blk, NW), lambda i: (i, 0)),
            pl.BlockSpec((blk, D), lambda i: (i, 0)),
            pl.BlockSpec((D, D), lambda i: (0, 0)),
            pl.BlockSpec((1, D), lambda i: (0, 0)),
        ],
        out_specs=pl.BlockSpec((blk, D), lambda i: (i, 0)),
        out_shape=jax.ShapeDtypeStruct((N, D), jnp.float32),
    )(agg_p, deg_t, x, W, b2)


def kernel(x, edge_index, W, b):
    src_r = edge_index[0].reshape(NW, NG, G)
    dst_r = edge_index[1].reshape(NW, NG, G)
    dst_f = edge_index[1].reshape(NW, EPW)
    agg_p, deg_p = _sc_aggregate(x, src_r, dst_r, dst_f)
    return _tc_finish(agg_p, deg_p.T, x, W, b.reshape(1, D))


# speedup vs baseline: 8.9329x; 8.9329x over previous
import jax
import jax.numpy as jnp
from jax import lax
from jax.experimental import pallas as pl
from jax.experimental.pallas import tpu as pltpu
from jax.experimental.pallas import tpu_sc as plsc

N = 10000
E = 320000
D = 128

NC = 2
NS = 16
NW = NC * NS
EPW = E // NW
G = 80
NG = EPW // G
NCH = N // G
CPS = -(-NCH // NS)


def _sc_body(x_hbm, src_hbm, dst_hbm,
             agg_hbm, deg_hbm,
             src_v, dst_v, rows_v, deg_v, acc_sh, sem):
    c = lax.axis_index("c")
    s = lax.axis_index("s")
    w = c * NS + s

    zeros16 = jnp.zeros((16,), jnp.float32)
    ones16 = jnp.ones((16,), jnp.float32)

    def zero_rows(i, _):
        for k in range(D // 16):
            rows_v[i, pl.ds(k * 16, 16)] = zeros16
        return 0
    lax.fori_loop(0, G, zero_rows, 0)

    def zero_deg(i, _):
        deg_v[pl.ds(i * 16, 16)] = zeros16
        return 0
    lax.fori_loop(0, N // 16, zero_deg, 0)

    def zero_acc(j, _):
        ch = s + j * NS

        @pl.when(ch < NCH)
        def _():
            pltpu.sync_copy(rows_v, acc_sh.at[pl.ds(ch * G, G), :])
        return 0
    lax.fori_loop(0, CPS, zero_acc, 0)

    pltpu.sync_copy(src_hbm.at[w], src_v)
    pltpu.sync_copy(dst_hbm.at[w], dst_v)

    def deg_step(g, _):
        for k in range(G // 16):
            idx16 = dst_v[g, pl.ds(k * 16, 16)]
            plsc.addupdate_scatter(deg_v, [idx16], ones16)
        return 0
    lax.fori_loop(0, NG, deg_step, 0)

    plsc.subcore_barrier()

    def edge_step(g, _):
        pltpu.async_copy(x_hbm.at[src_v.at[g]], rows_v, sem).wait()
        pltpu.sync_copy(rows_v, acc_sh.at[dst_v.at[g]], add=True)
        return 0
    lax.fori_loop(0, NG, edge_step, 0)

    plsc.subcore_barrier()

    def write_acc(j, _):
        ch = s + j * NS

        @pl.when(ch < NCH)
        def _():
            pltpu.sync_copy(acc_sh.at[pl.ds(ch * G, G), :], rows_v)
            pltpu.sync_copy(rows_v, agg_hbm.at[c, pl.ds(ch * G, G), :])
        return 0
    lax.fori_loop(0, CPS, write_acc, 0)
    pltpu.sync_copy(deg_v, deg_hbm.at[pl.ds(w * N, N)])


@jax.jit
def _sc_aggregate(x, src_r, dst_r):
    mesh = plsc.VectorSubcoreMesh(core_axis_name="c", subcore_axis_name="s")
    f = pl.kernel(
        _sc_body,
        out_type=(
            jax.ShapeDtypeStruct((NC, N, D), jnp.float32),
            jax.ShapeDtypeStruct((NW * N,), jnp.float32),
        ),
        mesh=mesh,
        compiler_params=pltpu.CompilerParams(
            use_tc_tiling_on_sc=False, needs_layout_passes=False),
        scratch_types=[
            pltpu.VMEM((NG, G), jnp.int32),
            pltpu.VMEM((NG, G), jnp.int32),
            pltpu.VMEM((G, D), jnp.float32),
            pltpu.VMEM((N,), jnp.float32),
            pltpu.VMEM_SHARED((N, D), jnp.float32),
            pltpu.SemaphoreType.DMA,
        ],
    )
    return f(x, src_r, dst_r)


def _tc_body(agg_ref, deg_ref, x_ref, w_ref, b_ref, o_ref):
    agg = agg_ref[0] + agg_ref[1] + x_ref[...]
    deg = jnp.sum(deg_ref[...], axis=1) + 1.0
    h = agg / deg[:, None]
    o_ref[...] = (
        jnp.dot(h, w_ref[...], preferred_element_type=jnp.float32) + b_ref[...]
    )


def _tc_finish(agg_p, deg_t, x, W, b2):
    blk = 2000
    grid = N // blk
    return pl.pallas_call(
        _tc_body,
        grid=(grid,),
        in_specs=[
            pl.BlockSpec((NC, blk, D), lambda i: (0, i, 0)),
            pl.BlockSpec((blk, NW), lambda i: (i, 0)),
            pl.BlockSpec((blk, D), lambda i: (i, 0)),
            pl.BlockSpec((D, D), lambda i: (0, 0)),
            pl.BlockSpec((1, D), lambda i: (0, 0)),
        ],
        out_specs=pl.BlockSpec((blk, D), lambda i: (i, 0)),
        out_shape=jax.ShapeDtypeStruct((N, D), jnp.float32),
    )(agg_p, deg_t, x, W, b2)


def kernel(x, edge_index, W, b):
    src_r = edge_index[0].reshape(NW, NG, G)
    dst_r = edge_index[1].reshape(NW, NG, G)
    agg_p, deg_f = _sc_aggregate(x, src_r, dst_r)
    deg_t = deg_f.reshape(NW, N).T
    return _tc_finish(agg_p, deg_t, x, W, b.reshape(1, D))


# speedup vs baseline: 13.5226x; 1.5138x over previous
import jax
import jax.numpy as jnp
from jax import lax
from jax.experimental import pallas as pl
from jax.experimental.pallas import tpu as pltpu
from jax.experimental.pallas import tpu_sc as plsc

N = 10000
E = 320000
D = 128

NC = 2
NS = 16
NW = NC * NS
EPW = E // NW
G = 80
NG = EPW // G
NCH = N // G
CPS = -(-NCH // NS)


def _sc_body(x_hbm, src_hbm, dst_hbm,
             agg_hbm, deg_hbm,
             src_v, dst_v, rows_v, deg_v, acc_sh, sem, sem2):
    c = lax.axis_index("c")
    s = lax.axis_index("s")
    w = c * NS + s

    zbuf = rows_v.at[0]

    zeros16 = jnp.zeros((16,), jnp.float32)
    ones16 = jnp.ones((16,), jnp.float32)

    def zero_rows(i, _):
        for k in range(D // 16):
            zbuf[i, pl.ds(k * 16, 16)] = zeros16
        return 0
    lax.fori_loop(0, G, zero_rows, 0)

    def zero_deg(i, _):
        deg_v[pl.ds(i * 16, 16)] = zeros16
        return 0
    lax.fori_loop(0, N // 16, zero_deg, 0)

    def zero_acc(j, _):
        ch = s + j * NS

        @pl.when(ch < NCH)
        def _():
            pltpu.sync_copy(zbuf, acc_sh.at[pl.ds(ch * G, G), :])
        return 0
    lax.fori_loop(0, CPS, zero_acc, 0)

    pltpu.sync_copy(src_hbm.at[w], src_v)
    pltpu.sync_copy(dst_hbm.at[w], dst_v)

    def deg_step(g, _):
        for k in range(G // 16):
            idx16 = dst_v[g, pl.ds(k * 16, 16)]
            plsc.addupdate_scatter(deg_v, [idx16], ones16)
        return 0
    lax.fori_loop(0, NG, deg_step, 0)

    plsc.subcore_barrier()

    rows_a = rows_v.at[0]
    rows_b = rows_v.at[1]

    def start(g, buf, s_):
        pltpu.async_copy(x_hbm.at[src_v.at[g]], buf, s_)

    def finish(g, buf, s_):
        pltpu.make_async_copy(x_hbm.at[src_v.at[g]], buf, s_).wait()
        pltpu.sync_copy(buf, acc_sh.at[dst_v.at[g]], add=True)

    start(0, rows_a, sem)

    def edge_pair(i, _):
        g = 2 * i
        start(g + 1, rows_b, sem2)
        finish(g, rows_a, sem)
        start(g + 2, rows_a, sem)
        finish(g + 1, rows_b, sem2)
        return 0
    lax.fori_loop(0, (NG - 1) // 2, edge_pair, 0)
    finish(NG - 1, rows_a, sem)

    plsc.subcore_barrier()

    def write_acc(j, _):
        ch = s + j * NS

        @pl.when(ch < NCH)
        def _():
            pltpu.sync_copy(acc_sh.at[pl.ds(ch * G, G), :], zbuf)
            pltpu.sync_copy(zbuf, agg_hbm.at[c, pl.ds(ch * G, G), :])
        return 0
    lax.fori_loop(0, CPS, write_acc, 0)
    pltpu.sync_copy(deg_v, deg_hbm.at[pl.ds(w * N, N)])


@jax.jit
def _sc_aggregate(x, src_r, dst_r):
    mesh = plsc.VectorSubcoreMesh(core_axis_name="c", subcore_axis_name="s")
    f = pl.kernel(
        _sc_body,
        out_type=(
            jax.ShapeDtypeStruct((NC, N, D), jnp.float32),
            jax.ShapeDtypeStruct((NW * N,), jnp.float32),
        ),
        mesh=mesh,
        compiler_params=pltpu.CompilerParams(
            use_tc_tiling_on_sc=False, needs_layout_passes=False),
        scratch_types=[
            pltpu.VMEM((NG, G), jnp.int32),
            pltpu.VMEM((NG, G), jnp.int32),
            pltpu.VMEM((2, G, D), jnp.float32),
            pltpu.VMEM((N,), jnp.float32),
            pltpu.VMEM_SHARED((N, D), jnp.float32),
            pltpu.SemaphoreType.DMA,
            pltpu.SemaphoreType.DMA,
        ],
    )
    return f(x, src_r, dst_r)


def _tc_body(agg_ref, deg_ref, x_ref, w_ref, b_ref, o_ref):
    agg = agg_ref[0] + agg_ref[1] + x_ref[...]
    deg = jnp.sum(deg_ref[...], axis=1) + 1.0
    h = agg / deg[:, None]
    o_ref[...] = (
        jnp.dot(h, w_ref[...], preferred_element_type=jnp.float32) + b_ref[...]
    )


def _tc_finish(agg_p, deg_t, x, W, b2):
    blk = 2000
    grid = N // blk
    return pl.pallas_call(
        _tc_body,
        grid=(grid,),
        in_specs=[
            pl.BlockSpec((NC, blk, D), lambda i: (0, i, 0)),
            pl.BlockSpec((blk, NW), lambda i: (i, 0)),
            pl.BlockSpec((blk, D), lambda i: (i, 0)),
            pl.BlockSpec((D, D), lambda i: (0, 0)),
            pl.BlockSpec((1, D), lambda i: (0, 0)),
        ],
        out_specs=pl.BlockSpec((blk, D), lambda i: (i, 0)),
        out_shape=jax.ShapeDtypeStruct((N, D), jnp.float32),
    )(agg_p, deg_t, x, W, b2)


def kernel(x, edge_index, W, b):
    src_r = edge_index[0].reshape(NW, NG, G)
    dst_r = edge_index[1].reshape(NW, NG, G)
    agg_p, deg_f = _sc_aggregate(x, src_r, dst_r)
    deg_t = deg_f.reshape(NW, N).T
    return _tc_finish(agg_p, deg_t, x, W, b.reshape(1, D))
